# baseline (speedup 1.0000x reference)
import jax
import jax.numpy as jnp
from jax.experimental import pallas as pl
from jax.experimental.pallas import tpu as pltpu

EPS = 1e-5
K_NEIGH = 16
M_OUT = 2048


def _ka_body(e_ref, w1_ref, x1_ref, s1_ref, q1_ref):
    i = pl.program_id(0)
    e = e_ref[...]
    x1 = jnp.dot(e, w1_ref[...].T, preferred_element_type=jnp.float32)
    x1_ref[...] = x1

    @pl.when(i == 0)
    def _init():
        s1_ref[...] = jnp.zeros_like(s1_ref)
        q1_ref[...] = jnp.zeros_like(q1_ref)

    s1_ref[...] += jnp.sum(x1, axis=0, keepdims=True)
    q1_ref[...] += jnp.sum(x1 * x1, axis=0, keepdims=True)


def _kb_body(x1_ref, s1_ref, q1_ref, g1_ref, b1_ref, w2_ref,
             ymax_ref, ymin_ref, s2_ref, q2_ref, *, count, mt):
    i = pl.program_id(0)
    mean1 = s1_ref[...] / count
    var1 = q1_ref[...] / count - mean1 * mean1
    scale1 = g1_ref[...] * jax.lax.rsqrt(var1 + EPS)
    shift1 = b1_ref[...] - mean1 * scale1
    h = jnp.maximum(x1_ref[...] * scale1 + shift1, 0.0)
    x2 = jnp.dot(h, w2_ref[...].T, preferred_element_type=jnp.float32)

    @pl.when(i == 0)
    def _init():
        s2_ref[...] = jnp.zeros_like(s2_ref)
        q2_ref[...] = jnp.zeros_like(q2_ref)

    s2_ref[...] += jnp.sum(x2, axis=0, keepdims=True)
    q2_ref[...] += jnp.sum(x2 * x2, axis=0, keepdims=True)
    x2r = x2.reshape(mt, K_NEIGH, 128)
    ymax_ref[...] = jnp.max(x2r, axis=1)
    ymin_ref[...] = jnp.min(x2r, axis=1)


def _kc_body(ymax_ref, ymin_ref, s2_ref, q2_ref, g2_ref, b2_ref, out_ref,
             *, count):
    mean2 = s2_ref[...] / count
    var2 = q2_ref[...] / count - mean2 * mean2
    scale2 = g2_ref[...] * jax.lax.rsqrt(var2 + EPS)
    shift2 = b2_ref[...] - mean2 * scale2
    z = jnp.maximum(ymax_ref[0] * scale2, ymin_ref[0] * scale2) + shift2
    out_ref[0] = jnp.maximum(z, 0.0).T


def _mlp_bn_max(edge, W1, gamma1, beta1, W2, gamma2, beta2, b):
    rows, c_in = edge.shape
    count = float(rows)
    RT = 4096
    x1, s1, q1 = pl.pallas_call(
        _ka_body,
        grid=(rows // RT,),
        in_specs=[
            pl.BlockSpec((RT, c_in), lambda i: (i, 0)),
            pl.BlockSpec((128, c_in), lambda i: (0, 0)),
        ],
        out_specs=[
            pl.BlockSpec((RT, 128), lambda i: (i, 0)),
            pl.BlockSpec((1, 128), lambda i: (0, 0)),
            pl.BlockSpec((1, 128), lambda i: (0, 0)),
        ],
        out_shape=[
            jax.ShapeDtypeStruct((rows, 128), jnp.float32),
            jax.ShapeDtypeStruct((1, 128), jnp.float32),
            jax.ShapeDtypeStruct((1, 128), jnp.float32),
        ],
    )(edge, W1)

    MT = 256
    bm = rows // K_NEIGH
    vec = lambda v: v.reshape(1, 128)
    ymax, ymin, s2, q2 = pl.pallas_call(
        lambda *refs: _kb_body(*refs, count=count, mt=MT),
        grid=(bm // MT,),
        in_specs=[
            pl.BlockSpec((MT * K_NEIGH, 128), lambda i: (i, 0)),
            pl.BlockSpec((1, 128), lambda i: (0, 0)),
            pl.BlockSpec((1, 128), lambda i: (0, 0)),
            pl.BlockSpec((1, 128), lambda i: (0, 0)),
            pl.BlockSpec((1, 128), lambda i: (0, 0)),
            pl.BlockSpec((128, 128), lambda i: (0, 0)),
        ],
        out_specs=[
            pl.BlockSpec((MT, 128), lambda i: (i, 0)),
            pl.BlockSpec((MT, 128), lambda i: (i, 0)),
            pl.BlockSpec((1, 128), lambda i: (0, 0)),
            pl.BlockSpec((1, 128), lambda i: (0, 0)),
        ],
        out_shape=[
            jax.ShapeDtypeStruct((bm, 128), jnp.float32),
            jax.ShapeDtypeStruct((bm, 128), jnp.float32),
            jax.ShapeDtypeStruct((1, 128), jnp.float32),
            jax.ShapeDtypeStruct((1, 128), jnp.float32),
        ],
    )(x1, s1, q1, vec(gamma1), vec(beta1), W2)

    m = bm // b
    ymax3 = ymax.reshape(b, m, 128)
    ymin3 = ymin.reshape(b, m, 128)
    MT2 = 512
    out = pl.pallas_call(
        lambda *refs: _kc_body(*refs, count=count),
        grid=(b, m // MT2),
        in_specs=[
            pl.BlockSpec((1, MT2, 128), lambda bi, mi: (bi, mi, 0)),
            pl.BlockSpec((1, MT2, 128), lambda bi, mi: (bi, mi, 0)),
            pl.BlockSpec((1, 128), lambda bi, mi: (0, 0)),
            pl.BlockSpec((1, 128), lambda bi, mi: (0, 0)),
            pl.BlockSpec((1, 128), lambda bi, mi: (0, 0)),
            pl.BlockSpec((1, 128), lambda bi, mi: (0, 0)),
        ],
        out_specs=pl.BlockSpec((1, 128, MT2), lambda bi, mi: (bi, 0, mi)),
        out_shape=jax.ShapeDtypeStruct((b, 128, m), jnp.float32),
    )(ymax3, ymin3, s2, q2, vec(gamma2), vec(beta2))
    return out


def kernel(xyz, feat, npoints, W1, gamma1, beta1, W2, gamma2, beta2):
    b, n, _ = xyz.shape
    c = feat.shape[1]
    m = M_OUT
    k = K_NEIGH
    sample_idx = (jnp.tile(jnp.arange(m, dtype=jnp.int32)[None, :], (b, 1))
                  + (jnp.asarray(npoints).astype(jnp.int32) - m))
    new_xyz = xyz[:, :m, :]

    feat_p = jnp.transpose(feat, (0, 2, 1))
    feat_cat = jnp.concatenate([feat_p, xyz], axis=-1)
    new_feat_cat = feat_cat[:, :m, :]
    d = (jnp.sum(new_xyz ** 2, axis=-1, keepdims=True)
         - 2.0 * jnp.einsum('bmd,bnd->bmn', new_xyz, xyz)
         + jnp.sum(xyz ** 2, axis=-1)[:, None, :])
    _, knn_idx = jax.lax.top_k(-d, k)
    knn_feat = jax.vmap(lambda f, i: f[i])(feat_cat, knn_idx)
    tiled = jnp.broadcast_to(new_feat_cat[:, :, None, :], knn_feat.shape)
    edge = jnp.concatenate([knn_feat - tiled, tiled], axis=-1)
    edge = edge.reshape(b * m * k, 2 * (c + 3))

    out_feat = _mlp_bn_max(edge, W1, gamma1, beta1, W2, gamma2, beta2, b)
    return new_xyz, out_feat, sample_idx.astype(jnp.int64)


# speedup vs baseline: 5.4847x; 5.4847x over previous
import jax
import jax.numpy as jnp
from jax.experimental import pallas as pl
from jax.experimental.pallas import tpu as pltpu

EPS = 1e-5
K_NEIGH = 16
M_OUT = 2048
BIG = 3.0e38


def _knn_body(q_ref, pt_ref, idx_ref, *, tm, n):
    ngrp = n // 128
    q = q_ref[0]
    pt = pt_ref[0]
    dot = jnp.dot(q, pt, preferred_element_type=jnp.float32)
    qq = jnp.sum(q * q, axis=1, keepdims=True)
    pp = jnp.sum(pt * pt, axis=0, keepdims=True)
    d = (qq - 2.0 * dot) + pp
    dg = d.reshape(tm, ngrp, 128)
    gmin = jnp.min(dg, axis=2)
    giota = jax.lax.broadcasted_iota(jnp.int32, (tm, ngrp), 1)
    gsel = []
    for _ in range(K_NEIGH):
        v = jnp.min(gmin, axis=1, keepdims=True)
        gi = jnp.min(jnp.where(gmin == v, giota, ngrp), axis=1, keepdims=True)
        gsel.append(gi)
        gmin = jnp.where(giota == gi, BIG, gmin)
    gsel = jnp.concatenate(gsel, axis=1)
    cand = jnp.broadcast_to(dg[:, 0:1, :], (tm, K_NEIGH, 128))
    for g in range(1, ngrp):
        cand = jnp.where(gsel[:, :, None] == g, dg[:, g:g + 1, :], cand)
    lane = jax.lax.broadcasted_iota(jnp.int32, (tm, K_NEIGH, 128), 2)
    gidx = gsel[:, :, None] * 128 + lane
    flat = cand.reshape(tm, K_NEIGH * 128)
    fidx = gidx.reshape(tm, K_NEIGH * 128)
    outs = []
    for _ in range(K_NEIGH):
        v = jnp.min(flat, axis=1, keepdims=True)
        ji = jnp.min(jnp.where(flat == v, fidx, jnp.int32(n)), axis=1,
                     keepdims=True)
        outs.append(ji)
        flat = jnp.where(fidx == ji, BIG, flat)
    idx_ref[0] = jnp.concatenate(outs, axis=1)


def _knn(xyz, b, n, m):
    TM = 256
    xyz_t = jnp.transpose(xyz, (0, 2, 1))
    return pl.pallas_call(
        lambda *refs: _knn_body(*refs, tm=TM, n=n),
        grid=(b, m // TM),
        in_specs=[
            pl.BlockSpec((1, TM, 3), lambda bi, mi: (bi, mi, 0)),
            pl.BlockSpec((1, 3, n), lambda bi, mi: (bi, 0, 0)),
        ],
        out_specs=pl.BlockSpec((1, TM, K_NEIGH), lambda bi, mi: (bi, mi, 0)),
        out_shape=jax.ShapeDtypeStruct((b, m, K_NEIGH), jnp.int32),
    )(xyz[:, :m, :], xyz_t)


def _ka_body(e_ref, w1_ref, x1_ref, s1_ref, q1_ref):
    i = pl.program_id(0)
    e = e_ref[...]
    x1 = jnp.dot(e, w1_ref[...].T, preferred_element_type=jnp.float32)
    x1_ref[...] = x1

    @pl.when(i == 0)
    def _init():
        s1_ref[...] = jnp.zeros_like(s1_ref)
        q1_ref[...] = jnp.zeros_like(q1_ref)

    s1_ref[...] += jnp.sum(x1, axis=0, keepdims=True)
    q1_ref[...] += jnp.sum(x1 * x1, axis=0, keepdims=True)


def _kb_body(x1_ref, s1_ref, q1_ref, g1_ref, b1_ref, w2_ref,
             ymax_ref, ymin_ref, s2_ref, q2_ref, *, count, mt):
    i = pl.program_id(0)
    mean1 = s1_ref[...] / count
    var1 = q1_ref[...] / count - mean1 * mean1
    scale1 = g1_ref[...] * jax.lax.rsqrt(var1 + EPS)
    shift1 = b1_ref[...] - mean1 * scale1
    h = jnp.maximum(x1_ref[...] * scale1 + shift1, 0.0)
    x2 = jnp.dot(h, w2_ref[...].T, preferred_element_type=jnp.float32)

    @pl.when(i == 0)
    def _init():
        s2_ref[...] = jnp.zeros_like(s2_ref)
        q2_ref[...] = jnp.zeros_like(q2_ref)

    s2_ref[...] += jnp.sum(x2, axis=0, keepdims=True)
    q2_ref[...] += jnp.sum(x2 * x2, axis=0, keepdims=True)
    x2r = x2.reshape(mt, K_NEIGH, 128)
    ymax_ref[...] = jnp.max(x2r, axis=1)
    ymin_ref[...] = jnp.min(x2r, axis=1)


def _kc_body(ymax_ref, ymin_ref, s2_ref, q2_ref, g2_ref, b2_ref, out_ref,
             *, count):
    mean2 = s2_ref[...] / count
    var2 = q2_ref[...] / count - mean2 * mean2
    scale2 = g2_ref[...] * jax.lax.rsqrt(var2 + EPS)
    shift2 = b2_ref[...] - mean2 * scale2
    z = jnp.maximum(ymax_ref[0] * scale2, ymin_ref[0] * scale2) + shift2
    out_ref[0] = jnp.maximum(z, 0.0).T


def _mlp_bn_max(edge, W1, gamma1, beta1, W2, gamma2, beta2, b):
    rows, c_in = edge.shape
    count = float(rows)
    RT = 4096
    x1, s1, q1 = pl.pallas_call(
        _ka_body,
        grid=(rows // RT,),
        in_specs=[
            pl.BlockSpec((RT, c_in), lambda i: (i, 0)),
            pl.BlockSpec((128, c_in), lambda i: (0, 0)),
        ],
        out_specs=[
            pl.BlockSpec((RT, 128), lambda i: (i, 0)),
            pl.BlockSpec((1, 128), lambda i: (0, 0)),
            pl.BlockSpec((1, 128), lambda i: (0, 0)),
        ],
        out_shape=[
            jax.ShapeDtypeStruct((rows, 128), jnp.float32),
            jax.ShapeDtypeStruct((1, 128), jnp.float32),
            jax.ShapeDtypeStruct((1, 128), jnp.float32),
        ],
    )(edge, W1)

    MT = 256
    bm = rows // K_NEIGH
    vec = lambda v: v.reshape(1, 128)
    ymax, ymin, s2, q2 = pl.pallas_call(
        lambda *refs: _kb_body(*refs, count=count, mt=MT),
        grid=(bm // MT,),
        in_specs=[
            pl.BlockSpec((MT * K_NEIGH, 128), lambda i: (i, 0)),
            pl.BlockSpec((1, 128), lambda i: (0, 0)),
            pl.BlockSpec((1, 128), lambda i: (0, 0)),
            pl.BlockSpec((1, 128), lambda i: (0, 0)),
            pl.BlockSpec((1, 128), lambda i: (0, 0)),
            pl.BlockSpec((128, 128), lambda i: (0, 0)),
        ],
        out_specs=[
            pl.BlockSpec((MT, 128), lambda i: (i, 0)),
            pl.BlockSpec((MT, 128), lambda i: (i, 0)),
            pl.BlockSpec((1, 128), lambda i: (0, 0)),
            pl.BlockSpec((1, 128), lambda i: (0, 0)),
        ],
        out_shape=[
            jax.ShapeDtypeStruct((bm, 128), jnp.float32),
            jax.ShapeDtypeStruct((bm, 128), jnp.float32),
            jax.ShapeDtypeStruct((1, 128), jnp.float32),
            jax.ShapeDtypeStruct((1, 128), jnp.float32),
        ],
    )(x1, s1, q1, vec(gamma1), vec(beta1), W2)

    m = bm // b
    ymax3 = ymax.reshape(b, m, 128)
    ymin3 = ymin.reshape(b, m, 128)
    MT2 = 512
    out = pl.pallas_call(
        lambda *refs: _kc_body(*refs, count=count),
        grid=(b, m // MT2),
        in_specs=[
            pl.BlockSpec((1, MT2, 128), lambda bi, mi: (bi, mi, 0)),
            pl.BlockSpec((1, MT2, 128), lambda bi, mi: (bi, mi, 0)),
            pl.BlockSpec((1, 128), lambda bi, mi: (0, 0)),
            pl.BlockSpec((1, 128), lambda bi, mi: (0, 0)),
            pl.BlockSpec((1, 128), lambda bi, mi: (0, 0)),
            pl.BlockSpec((1, 128), lambda bi, mi: (0, 0)),
        ],
        out_specs=pl.BlockSpec((1, 128, MT2), lambda bi, mi: (bi, 0, mi)),
        out_shape=jax.ShapeDtypeStruct((b, 128, m), jnp.float32),
    )(ymax3, ymin3, s2, q2, vec(gamma2), vec(beta2))
    return out


def kernel(xyz, feat, npoints, W1, gamma1, beta1, W2, gamma2, beta2):
    b, n, _ = xyz.shape
    c = feat.shape[1]
    m = M_OUT
    k = K_NEIGH
    sample_idx = (jnp.tile(jnp.arange(m, dtype=jnp.int32)[None, :], (b, 1))
                  + (jnp.asarray(npoints).astype(jnp.int32) - m))
    new_xyz = xyz[:, :m, :]

    feat_p = jnp.transpose(feat, (0, 2, 1))
    feat_cat = jnp.concatenate([feat_p, xyz], axis=-1)
    new_feat_cat = feat_cat[:, :m, :]
    knn_idx = _knn(xyz, b, n, m)
    knn_feat = jax.vmap(lambda f, i: f[i])(feat_cat, knn_idx)
    tiled = jnp.broadcast_to(new_feat_cat[:, :, None, :], knn_feat.shape)
    edge = jnp.concatenate([knn_feat - tiled, tiled], axis=-1)
    edge = edge.reshape(b * m * k, 2 * (c + 3))

    out_feat = _mlp_bn_max(edge, W1, gamma1, beta1, W2, gamma2, beta2, b)
    return new_xyz, out_feat, sample_idx.astype(jnp.int64)


# speedup vs baseline: 11.6630x; 2.1265x over previous
import jax
import jax.numpy as jnp
from jax.experimental import pallas as pl
from jax.experimental.pallas import tpu as pltpu

EPS = 1e-5
K_NEIGH = 16
M_OUT = 2048
BIG = 3.0e38


def _knn_body(q_ref, pt_ref, idx_ref, *, tm, n):
    ngrp = n // 128
    q = q_ref[0]
    pt = pt_ref[0]
    dot = jnp.dot(q, pt, preferred_element_type=jnp.float32)
    qq = jnp.sum(q * q, axis=1, keepdims=True)
    pp = jnp.sum(pt * pt, axis=0, keepdims=True)
    d = (qq - 2.0 * dot) + pp
    dg = d.reshape(tm, ngrp, 128)
    gmin = jnp.min(dg, axis=2)
    giota = jax.lax.broadcasted_iota(jnp.int32, (tm, ngrp), 1)
    gsel = []
    for _ in range(K_NEIGH):
        v = jnp.min(gmin, axis=1, keepdims=True)
        gi = jnp.min(jnp.where(gmin == v, giota, ngrp), axis=1, keepdims=True)
        gsel.append(gi)
        gmin = jnp.where(giota == gi, BIG, gmin)
    gsel = jnp.concatenate(gsel, axis=1)
    cand = jnp.broadcast_to(dg[:, 0:1, :], (tm, K_NEIGH, 128))
    for g in range(1, ngrp):
        cand = jnp.where(gsel[:, :, None] == g, dg[:, g:g + 1, :], cand)
    lane = jax.lax.broadcasted_iota(jnp.int32, (tm, K_NEIGH, 128), 2)
    gidx = gsel[:, :, None] * 128 + lane
    flat = cand.reshape(tm, K_NEIGH * 128)
    fidx = gidx.reshape(tm, K_NEIGH * 128)
    outs = []
    for _ in range(K_NEIGH):
        v = jnp.min(flat, axis=1, keepdims=True)
        ji = jnp.min(jnp.where(flat == v, fidx, jnp.int32(n)), axis=1,
                     keepdims=True)
        outs.append(ji)
        flat = jnp.where(fidx == ji, BIG, flat)
    idx_ref[0] = jnp.concatenate(outs, axis=1)


def _knn(xyz, b, n, m):
    TM = 256
    xyz_t = jnp.transpose(xyz, (0, 2, 1))
    return pl.pallas_call(
        lambda *refs: _knn_body(*refs, tm=TM, n=n),
        grid=(b, m // TM),
        in_specs=[
            pl.BlockSpec((1, TM, 3), lambda bi, mi: (bi, mi, 0)),
            pl.BlockSpec((1, 3, n), lambda bi, mi: (bi, 0, 0)),
        ],
        out_specs=pl.BlockSpec((1, TM, K_NEIGH), lambda bi, mi: (bi, mi, 0)),
        out_shape=jax.ShapeDtypeStruct((b, m, K_NEIGH), jnp.int32),
    )(xyz[:, :m, :], xyz_t)


def _proj_body(feat_ref, xyz_ref, w_ref, g_ref):
    ft = feat_ref[0].T
    fc = jnp.concatenate([ft, xyz_ref[0]], axis=1)
    g_ref[0] = jnp.dot(fc, w_ref[...], preferred_element_type=jnp.float32)


def _proj(feat, xyz, w, b, n, tn):
    c = feat.shape[1]
    return pl.pallas_call(
        _proj_body,
        grid=(b, n // tn),
        in_specs=[
            pl.BlockSpec((1, c, tn), lambda bi, ni: (bi, 0, ni)),
            pl.BlockSpec((1, tn, 3), lambda bi, ni: (bi, ni, 0)),
            pl.BlockSpec((c + 3, 128), lambda bi, ni: (0, 0)),
        ],
        out_specs=pl.BlockSpec((1, tn, 128), lambda bi, ni: (bi, ni, 0)),
        out_shape=jax.ShapeDtypeStruct((b, n, 128), jnp.float32),
    )(feat, xyz, w)


def _ks_body(gg_ref, h_ref, s1_ref, q1_ref, *, mt):
    i = pl.program_id(0)
    x1 = gg_ref[...].reshape(mt, K_NEIGH, 128) + h_ref[...][:, None, :]

    @pl.when(i == 0)
    def _init():
        s1_ref[...] = jnp.zeros_like(s1_ref)
        q1_ref[...] = jnp.zeros_like(q1_ref)

    s1_ref[...] += jnp.sum(x1, axis=(0, 1)).reshape(1, 128)
    q1_ref[...] += jnp.sum(x1 * x1, axis=(0, 1)).reshape(1, 128)


def _kb_body(gg_ref, h_ref, s1_ref, q1_ref, g1_ref, b1_ref, w2_ref,
             ymax_ref, ymin_ref, s2_ref, q2_ref, *, count, mt):
    i = pl.program_id(0)
    mean1 = s1_ref[...] / count
    var1 = q1_ref[...] / count - mean1 * mean1
    scale1 = g1_ref[...] * jax.lax.rsqrt(var1 + EPS)
    shift1 = b1_ref[...] - mean1 * scale1
    x1 = (gg_ref[...].reshape(mt, K_NEIGH, 128)
          + h_ref[...][:, None, :]).reshape(mt * K_NEIGH, 128)
    h = jnp.maximum(x1 * scale1 + shift1, 0.0)
    x2 = jnp.dot(h, w2_ref[...].T, preferred_element_type=jnp.float32)

    @pl.when(i == 0)
    def _init():
        s2_ref[...] = jnp.zeros_like(s2_ref)
        q2_ref[...] = jnp.zeros_like(q2_ref)

    s2_ref[...] += jnp.sum(x2, axis=0, keepdims=True)
    q2_ref[...] += jnp.sum(x2 * x2, axis=0, keepdims=True)
    x2r = x2.reshape(mt, K_NEIGH, 128)
    ymax_ref[...] = jnp.max(x2r, axis=1)
    ymin_ref[...] = jnp.min(x2r, axis=1)


def _kc_body(ymax_ref, ymin_ref, s2_ref, q2_ref, g2_ref, b2_ref, out_ref,
             *, count):
    mean2 = s2_ref[...] / count
    var2 = q2_ref[...] / count - mean2 * mean2
    scale2 = g2_ref[...] * jax.lax.rsqrt(var2 + EPS)
    shift2 = b2_ref[...] - mean2 * scale2
    z = jnp.maximum(ymax_ref[0] * scale2, ymin_ref[0] * scale2) + shift2
    out_ref[0] = jnp.maximum(z, 0.0).T


def _mlp_bn_max(gg, h, W2, gamma1, beta1, gamma2, beta2, b):
    rows = gg.shape[0]
    count = float(rows)
    bm = rows // K_NEIGH
    MT = 256
    vec = lambda v: v.reshape(1, 128)
    s1, q1 = pl.pallas_call(
        lambda *refs: _ks_body(*refs, mt=MT),
        grid=(bm // MT,),
        in_specs=[
            pl.BlockSpec((MT * K_NEIGH, 128), lambda i: (i, 0)),
            pl.BlockSpec((MT, 128), lambda i: (i, 0)),
        ],
        out_specs=[
            pl.BlockSpec((1, 128), lambda i: (0, 0)),
            pl.BlockSpec((1, 128), lambda i: (0, 0)),
        ],
        out_shape=[
            jax.ShapeDtypeStruct((1, 128), jnp.float32),
            jax.ShapeDtypeStruct((1, 128), jnp.float32),
        ],
    )(gg, h)

    ymax, ymin, s2, q2 = pl.pallas_call(
        lambda *refs: _kb_body(*refs, count=count, mt=MT),
        grid=(bm // MT,),
        in_specs=[
            pl.BlockSpec((MT * K_NEIGH, 128), lambda i: (i, 0)),
            pl.BlockSpec((MT, 128), lambda i: (i, 0)),
            pl.BlockSpec((1, 128), lambda i: (0, 0)),
            pl.BlockSpec((1, 128), lambda i: (0, 0)),
            pl.BlockSpec((1, 128), lambda i: (0, 0)),
            pl.BlockSpec((1, 128), lambda i: (0, 0)),
            pl.BlockSpec((128, 128), lambda i: (0, 0)),
        ],
        out_specs=[
            pl.BlockSpec((MT, 128), lambda i: (i, 0)),
            pl.BlockSpec((MT, 128), lambda i: (i, 0)),
            pl.BlockSpec((1, 128), lambda i: (0, 0)),
            pl.BlockSpec((1, 128), lambda i: (0, 0)),
        ],
        out_shape=[
            jax.ShapeDtypeStruct((bm, 128), jnp.float32),
            jax.ShapeDtypeStruct((bm, 128), jnp.float32),
            jax.ShapeDtypeStruct((1, 128), jnp.float32),
            jax.ShapeDtypeStruct((1, 128), jnp.float32),
        ],
    )(gg, h, s1, q1, vec(gamma1), vec(beta1), W2)

    m = bm // b
    ymax3 = ymax.reshape(b, m, 128)
    ymin3 = ymin.reshape(b, m, 128)
    MT2 = 512
    out = pl.pallas_call(
        lambda *refs: _kc_body(*refs, count=count),
        grid=(b, m // MT2),
        in_specs=[
            pl.BlockSpec((1, MT2, 128), lambda bi, mi: (bi, mi, 0)),
            pl.BlockSpec((1, MT2, 128), lambda bi, mi: (bi, mi, 0)),
            pl.BlockSpec((1, 128), lambda bi, mi: (0, 0)),
            pl.BlockSpec((1, 128), lambda bi, mi: (0, 0)),
            pl.BlockSpec((1, 128), lambda bi, mi: (0, 0)),
            pl.BlockSpec((1, 128), lambda bi, mi: (0, 0)),
        ],
        out_specs=pl.BlockSpec((1, 128, MT2), lambda bi, mi: (bi, 0, mi)),
        out_shape=jax.ShapeDtypeStruct((b, 128, m), jnp.float32),
    )(ymax3, ymin3, s2, q2, vec(gamma2), vec(beta2))
    return out


def kernel(xyz, feat, npoints, W1, gamma1, beta1, W2, gamma2, beta2):
    b, n, _ = xyz.shape
    c = feat.shape[1]
    m = M_OUT
    k = K_NEIGH
    sample_idx = (jnp.tile(jnp.arange(m, dtype=jnp.int32)[None, :], (b, 1))
                  + (jnp.asarray(npoints).astype(jnp.int32) - m))
    new_xyz = xyz[:, :m, :]

    knn_idx = _knn(xyz, b, n, m)

    w1a_t = jnp.transpose(W1[:, :c + 3])
    wd_t = jnp.transpose(W1[:, c + 3:]) - w1a_t
    G = _proj(feat, xyz, w1a_t, b, n, 512)
    H = _proj(feat[:, :, :m], xyz[:, :m, :], wd_t, b, m, 512)

    idx_flat = (knn_idx + (jnp.arange(b, dtype=jnp.int32) * n)[:, None, None])
    gg = G.reshape(b * n, 128)[idx_flat.reshape(-1)]
    h = H.reshape(b * m, 128)

    out_feat = _mlp_bn_max(gg, h, W2, gamma1, beta1, gamma2, beta2, b)
    return new_xyz, out_feat, sample_idx.astype(jnp.int64)


# speedup vs baseline: 14.4834x; 1.2418x over previous
import functools

import jax
import jax.numpy as jnp
from jax import lax
from jax.experimental import pallas as pl
from jax.experimental.pallas import tpu as pltpu
from jax.experimental.pallas import tpu_sc as plsc

EPS = 1e-5
K_NEIGH = 16
M_OUT = 2048
BIG = 3.0e38


def _knn_body(q_ref, pt_ref, idx_ref, *, tm, n):
    ngrp = n // 128
    q = q_ref[0]
    pt = pt_ref[0]
    dot = jnp.dot(q, pt, preferred_element_type=jnp.float32)
    qq = jnp.sum(q * q, axis=1, keepdims=True)
    pp = jnp.sum(pt * pt, axis=0, keepdims=True)
    d = (qq - 2.0 * dot) + pp
    dg = d.reshape(tm, ngrp, 128)
    gmin = jnp.min(dg, axis=2)
    giota = jax.lax.broadcasted_iota(jnp.int32, (tm, ngrp), 1)
    gsel = []
    for _ in range(K_NEIGH):
        v = jnp.min(gmin, axis=1, keepdims=True)
        gi = jnp.min(jnp.where(gmin == v, giota, ngrp), axis=1, keepdims=True)
        gsel.append(gi)
        gmin = jnp.where(giota == gi, BIG, gmin)
    gsel = jnp.concatenate(gsel, axis=1)
    cand = jnp.broadcast_to(dg[:, 0:1, :], (tm, K_NEIGH, 128))
    for g in range(1, ngrp):
        cand = jnp.where(gsel[:, :, None] == g, dg[:, g:g + 1, :], cand)
    lane = jax.lax.broadcasted_iota(jnp.int32, (tm, K_NEIGH, 128), 2)
    gidx = gsel[:, :, None] * 128 + lane
    flat = cand.reshape(tm, K_NEIGH * 128)
    fidx = gidx.reshape(tm, K_NEIGH * 128)
    outs = []
    for _ in range(K_NEIGH):
        v = jnp.min(flat, axis=1, keepdims=True)
        ji = jnp.min(jnp.where(flat == v, fidx, jnp.int32(n)), axis=1,
                     keepdims=True)
        outs.append(ji)
        flat = jnp.where(fidx == ji, BIG, flat)
    idx_ref[0] = jnp.concatenate(outs, axis=1)


def _knn(xyz, b, n, m):
    TM = 256
    xyz_t = jnp.transpose(xyz, (0, 2, 1))
    return pl.pallas_call(
        lambda *refs: _knn_body(*refs, tm=TM, n=n),
        grid=(b, m // TM),
        in_specs=[
            pl.BlockSpec((1, TM, 3), lambda bi, mi: (bi, mi, 0)),
            pl.BlockSpec((1, 3, n), lambda bi, mi: (bi, 0, 0)),
        ],
        out_specs=pl.BlockSpec((1, TM, K_NEIGH), lambda bi, mi: (bi, mi, 0)),
        out_shape=jax.ShapeDtypeStruct((b, m, K_NEIGH), jnp.int32),
    )(xyz[:, :m, :], xyz_t)


def _proj_body(feat_ref, xyz_ref, w_ref, g_ref):
    ft = feat_ref[0].T
    fc = jnp.concatenate([ft, xyz_ref[0]], axis=1)
    g_ref[0] = jnp.dot(fc, w_ref[...], preferred_element_type=jnp.float32)


def _proj(feat, xyz, w, b, n, tn):
    c = feat.shape[1]
    return pl.pallas_call(
        _proj_body,
        grid=(b, n // tn),
        in_specs=[
            pl.BlockSpec((1, c, tn), lambda bi, ni: (bi, 0, ni)),
            pl.BlockSpec((1, tn, 3), lambda bi, ni: (bi, ni, 0)),
            pl.BlockSpec((c + 3, 128), lambda bi, ni: (0, 0)),
        ],
        out_specs=pl.BlockSpec((1, tn, 128), lambda bi, ni: (bi, ni, 0)),
        out_shape=jax.ShapeDtypeStruct((b, n, 128), jnp.float32),
    )(feat, xyz, w)


def _ks_body(gg_ref, h_ref, s1_ref, q1_ref, *, mt):
    i = pl.program_id(0)
    x1 = gg_ref[...].reshape(mt, K_NEIGH, 128) + h_ref[...][:, None, :]

    @pl.when(i == 0)
    def _init():
        s1_ref[...] = jnp.zeros_like(s1_ref)
        q1_ref[...] = jnp.zeros_like(q1_ref)

    s1_ref[...] += jnp.sum(x1, axis=(0, 1)).reshape(1, 128)
    q1_ref[...] += jnp.sum(x1 * x1, axis=(0, 1)).reshape(1, 128)


def _kb_body(gg_ref, h_ref, s1_ref, q1_ref, g1_ref, b1_ref, w2_ref,
             ymax_ref, ymin_ref, s2_ref, q2_ref, *, count, mt):
    i = pl.program_id(0)
    mean1 = s1_ref[...] / count
    var1 = q1_ref[...] / count - mean1 * mean1
    scale1 = g1_ref[...] * jax.lax.rsqrt(var1 + EPS)
    shift1 = b1_ref[...] - mean1 * scale1
    x1 = (gg_ref[...].reshape(mt, K_NEIGH, 128)
          + h_ref[...][:, None, :]).reshape(mt * K_NEIGH, 128)
    h = jnp.maximum(x1 * scale1 + shift1, 0.0)
    x2 = jnp.dot(h, w2_ref[...].T, preferred_element_type=jnp.float32)

    @pl.when(i == 0)
    def _init():
        s2_ref[...] = jnp.zeros_like(s2_ref)
        q2_ref[...] = jnp.zeros_like(q2_ref)

    s2_ref[...] += jnp.sum(x2, axis=0, keepdims=True)
    q2_ref[...] += jnp.sum(x2 * x2, axis=0, keepdims=True)
    x2r = x2.reshape(mt, K_NEIGH, 128)
    ymax_ref[...] = jnp.max(x2r, axis=1)
    ymin_ref[...] = jnp.min(x2r, axis=1)


def _kc_body(ymax_ref, ymin_ref, s2_ref, q2_ref, g2_ref, b2_ref, out_ref,
             *, count):
    mean2 = s2_ref[...] / count
    var2 = q2_ref[...] / count - mean2 * mean2
    scale2 = g2_ref[...] * jax.lax.rsqrt(var2 + EPS)
    shift2 = b2_ref[...] - mean2 * scale2
    z = jnp.maximum(ymax_ref[0] * scale2, ymin_ref[0] * scale2) + shift2
    out_ref[0] = jnp.maximum(z, 0.0).T


def _sc_gather(table, idx):
    v, dimw = table.shape
    bsz = idx.shape[0]
    nw = 32
    per_w = bsz // nw
    chunk = 128
    nchunk = per_w // chunk

    @functools.partial(
        pl.kernel,
        out_type=jax.ShapeDtypeStruct((bsz, dimw), jnp.float32),
        mesh=plsc.VectorSubcoreMesh(core_axis_name="c", subcore_axis_name="s"),
        scratch_types=[
            pltpu.VMEM((2, chunk), jnp.int32),
            pltpu.VMEM((2, chunk, dimw), jnp.float32),
            pltpu.SemaphoreType.DMA,
            pltpu.SemaphoreType.DMA,
        ],
    )
    def k(table_hbm, idx_hbm, out_hbm, idx_v, rows_v, gsem, osem):
        wid = lax.axis_index("s") * 2 + lax.axis_index("c")
        base = wid * per_w

        def body(i, carry):
            slot = lax.rem(i, 2)
            pltpu.sync_copy(idx_hbm.at[pl.ds(base + i * chunk, chunk)],
                            idx_v.at[slot])
            cp = pltpu.async_copy(table_hbm.at[idx_v.at[slot]],
                                  rows_v.at[slot], gsem)
            cp.wait()
            ocp = pltpu.async_copy(rows_v.at[slot],
                                   out_hbm.at[pl.ds(base + i * chunk, chunk)],
                                   osem)
            ocp.wait()
            return carry

        lax.fori_loop(0, nchunk, body, 0)

    return k(table, idx)


def _mlp_bn_max(gg, h, W2, gamma1, beta1, gamma2, beta2, b):
    rows = gg.shape[0]
    count = float(rows)
    bm = rows // K_NEIGH
    MT = 256
    vec = lambda v: v.reshape(1, 128)
    s1, q1 = pl.pallas_call(
        lambda *refs: _ks_body(*refs, mt=MT),
        grid=(bm // MT,),
        in_specs=[
            pl.BlockSpec((MT * K_NEIGH, 128), lambda i: (i, 0)),
            pl.BlockSpec((MT, 128), lambda i: (i, 0)),
        ],
        out_specs=[
            pl.BlockSpec((1, 128), lambda i: (0, 0)),
            pl.BlockSpec((1, 128), lambda i: (0, 0)),
        ],
        out_shape=[
            jax.ShapeDtypeStruct((1, 128), jnp.float32),
            jax.ShapeDtypeStruct((1, 128), jnp.float32),
        ],
    )(gg, h)

    ymax, ymin, s2, q2 = pl.pallas_call(
        lambda *refs: _kb_body(*refs, count=count, mt=MT),
        grid=(bm // MT,),
        in_specs=[
            pl.BlockSpec((MT * K_NEIGH, 128), lambda i: (i, 0)),
            pl.BlockSpec((MT, 128), lambda i: (i, 0)),
            pl.BlockSpec((1, 128), lambda i: (0, 0)),
            pl.BlockSpec((1, 128), lambda i: (0, 0)),
            pl.BlockSpec((1, 128), lambda i: (0, 0)),
            pl.BlockSpec((1, 128), lambda i: (0, 0)),
            pl.BlockSpec((128, 128), lambda i: (0, 0)),
        ],
        out_specs=[
            pl.BlockSpec((MT, 128), lambda i: (i, 0)),
            pl.BlockSpec((MT, 128), lambda i: (i, 0)),
            pl.BlockSpec((1, 128), lambda i: (0, 0)),
            pl.BlockSpec((1, 128), lambda i: (0, 0)),
        ],
        out_shape=[
            jax.ShapeDtypeStruct((bm, 128), jnp.float32),
            jax.ShapeDtypeStruct((bm, 128), jnp.float32),
            jax.ShapeDtypeStruct((1, 128), jnp.float32),
            jax.ShapeDtypeStruct((1, 128), jnp.float32),
        ],
    )(gg, h, s1, q1, vec(gamma1), vec(beta1), W2)

    m = bm // b
    ymax3 = ymax.reshape(b, m, 128)
    ymin3 = ymin.reshape(b, m, 128)
    MT2 = 512
    out = pl.pallas_call(
        lambda *refs: _kc_body(*refs, count=count),
        grid=(b, m // MT2),
        in_specs=[
            pl.BlockSpec((1, MT2, 128), lambda bi, mi: (bi, mi, 0)),
            pl.BlockSpec((1, MT2, 128), lambda bi, mi: (bi, mi, 0)),
            pl.BlockSpec((1, 128), lambda bi, mi: (0, 0)),
            pl.BlockSpec((1, 128), lambda bi, mi: (0, 0)),
            pl.BlockSpec((1, 128), lambda bi, mi: (0, 0)),
            pl.BlockSpec((1, 128), lambda bi, mi: (0, 0)),
        ],
        out_specs=pl.BlockSpec((1, 128, MT2), lambda bi, mi: (bi, 0, mi)),
        out_shape=jax.ShapeDtypeStruct((b, 128, m), jnp.float32),
    )(ymax3, ymin3, s2, q2, vec(gamma2), vec(beta2))
    return out


def kernel(xyz, feat, npoints, W1, gamma1, beta1, W2, gamma2, beta2):
    b, n, _ = xyz.shape
    c = feat.shape[1]
    m = M_OUT
    k = K_NEIGH
    sample_idx = (jnp.tile(jnp.arange(m, dtype=jnp.int32)[None, :], (b, 1))
                  + (jnp.asarray(npoints).astype(jnp.int32) - m))
    new_xyz = xyz[:, :m, :]

    knn_idx = _knn(xyz, b, n, m)

    w1a_t = jnp.transpose(W1[:, :c + 3])
    wd_t = jnp.transpose(W1[:, c + 3:]) - w1a_t
    G = _proj(feat, xyz, w1a_t, b, n, 512)
    H = _proj(feat[:, :, :m], xyz[:, :m, :], wd_t, b, m, 512)

    idx_flat = (knn_idx + (jnp.arange(b, dtype=jnp.int32) * n)[:, None, None])
    gg = _sc_gather(G.reshape(b * n, 128), idx_flat.reshape(-1))
    h = H.reshape(b * m, 128)

    out_feat = _mlp_bn_max(gg, h, W2, gamma1, beta1, gamma2, beta2, b)
    return new_xyz, out_feat, sample_idx.astype(jnp.int64)


# speedup vs baseline: 19.7901x; 1.3664x over previous
import functools

import jax
import jax.numpy as jnp
from jax import lax
from jax.experimental import pallas as pl
from jax.experimental.pallas import tpu as pltpu
from jax.experimental.pallas import tpu_sc as plsc

EPS = 1e-5
K_NEIGH = 16
M_OUT = 2048
BIG = 3.0e38


def _knn_body(p_ref, qt_ref, idx_ref, *, tq, n):
    ngrp = n // 8
    p = p_ref[0]
    qt = qt_ref[0]
    dot = jnp.dot(p, qt, preferred_element_type=jnp.float32)
    qq = jnp.sum(qt * qt, axis=0, keepdims=True)
    pp = jnp.sum(p * p, axis=1, keepdims=True)
    d = (qq - 2.0 * dot) + pp
    d3 = d.reshape(ngrp, 8, tq)
    gmin = jnp.min(d3, axis=1)
    giota = jax.lax.broadcasted_iota(jnp.int32, (ngrp, tq), 0)
    gsel = []
    for _ in range(K_NEIGH):
        v = jnp.min(gmin, axis=0, keepdims=True)
        gi = jnp.min(jnp.where(gmin == v, giota, ngrp), axis=0, keepdims=True)
        gsel.append(gi)
        gmin = jnp.where(giota == gi, BIG, gmin)
    nbit = max(1, (ngrp - 1).bit_length())
    cands = []
    for t in range(K_NEIGH):
        gi = gsel[t]
        x = d3
        for bit in reversed(range(nbit)):
            half = x.shape[0] // 2
            take_hi = ((gi >> bit) & 1) == 1
            x = jnp.where(take_hi[:, None, :], x[half:], x[:half])
        cands.append(x)
    cand = jnp.concatenate(cands, axis=0)
    gsel_arr = jnp.concatenate(gsel, axis=0)
    sub = jax.lax.broadcasted_iota(jnp.int32, (K_NEIGH, 8, tq), 1)
    gidx = gsel_arr[:, None, :] * 8 + sub
    flat = cand.reshape(K_NEIGH * 8, tq)
    fidx = gidx.reshape(K_NEIGH * 8, tq)
    outs = []
    for _ in range(K_NEIGH):
        v = jnp.min(flat, axis=0, keepdims=True)
        ji = jnp.min(jnp.where(flat == v, fidx, jnp.int32(n)), axis=0,
                     keepdims=True)
        outs.append(ji)
        flat = jnp.where(fidx == ji, BIG, flat)
    idx_ref[0] = jnp.concatenate(outs, axis=0)


def _knn(xyz, b, n, m):
    TQ = 128
    xyz_t = jnp.transpose(xyz[:, :m, :], (0, 2, 1))
    return pl.pallas_call(
        lambda *refs: _knn_body(*refs, tq=TQ, n=n),
        grid=(b, m // TQ),
        in_specs=[
            pl.BlockSpec((1, n, 3), lambda bi, mi: (bi, 0, 0)),
            pl.BlockSpec((1, 3, TQ), lambda bi, mi: (bi, 0, mi)),
        ],
        out_specs=pl.BlockSpec((1, K_NEIGH, TQ), lambda bi, mi: (bi, 0, mi)),
        out_shape=jax.ShapeDtypeStruct((b, K_NEIGH, m), jnp.int32),
    )(xyz, xyz_t)


def _proj_body(feat_ref, xyz_ref, w_ref, g_ref):
    ft = feat_ref[0].T
    fc = jnp.concatenate([ft, xyz_ref[0]], axis=1)
    g_ref[0] = jnp.dot(fc, w_ref[...], preferred_element_type=jnp.float32)


def _proj(feat, xyz, w, b, n, tn):
    c = feat.shape[1]
    return pl.pallas_call(
        _proj_body,
        grid=(b, n // tn),
        in_specs=[
            pl.BlockSpec((1, c, tn), lambda bi, ni: (bi, 0, ni)),
            pl.BlockSpec((1, tn, 3), lambda bi, ni: (bi, ni, 0)),
            pl.BlockSpec((c + 3, 128), lambda bi, ni: (0, 0)),
        ],
        out_specs=pl.BlockSpec((1, tn, 128), lambda bi, ni: (bi, ni, 0)),
        out_shape=jax.ShapeDtypeStruct((b, n, 128), jnp.float32),
    )(feat, xyz, w)


def _ks_body(gg_ref, h_ref, s1_ref, q1_ref, *, mt):
    first = (pl.program_id(0) == 0) & (pl.program_id(1) == 0)
    x1 = gg_ref[0] + h_ref[0][None, :, :]

    @pl.when(first)
    def _init():
        s1_ref[...] = jnp.zeros_like(s1_ref)
        q1_ref[...] = jnp.zeros_like(q1_ref)

    s1_ref[...] += jnp.sum(x1, axis=(0, 1)).reshape(1, 128)
    q1_ref[...] += jnp.sum(x1 * x1, axis=(0, 1)).reshape(1, 128)


def _kb_body(gg_ref, h_ref, s1_ref, q1_ref, g1_ref, b1_ref, w2_ref,
             ymax_ref, ymin_ref, s2_ref, q2_ref, *, count, mt):
    first = (pl.program_id(0) == 0) & (pl.program_id(1) == 0)
    mean1 = s1_ref[...] / count
    var1 = q1_ref[...] / count - mean1 * mean1
    scale1 = g1_ref[...] * jax.lax.rsqrt(var1 + EPS)
    shift1 = b1_ref[...] - mean1 * scale1
    x1 = (gg_ref[0] + h_ref[0][None, :, :]).reshape(K_NEIGH * mt, 128)
    h = jnp.maximum(x1 * scale1 + shift1, 0.0)
    x2 = jnp.dot(h, w2_ref[...].T, preferred_element_type=jnp.float32)

    @pl.when(first)
    def _init():
        s2_ref[...] = jnp.zeros_like(s2_ref)
        q2_ref[...] = jnp.zeros_like(q2_ref)

    s2_ref[...] += jnp.sum(x2, axis=0, keepdims=True)
    q2_ref[...] += jnp.sum(x2 * x2, axis=0, keepdims=True)
    x2r = x2.reshape(K_NEIGH, mt, 128)
    ymax_ref[...] = jnp.max(x2r, axis=0)
    ymin_ref[...] = jnp.min(x2r, axis=0)


def _kc_body(ymax_ref, ymin_ref, s2_ref, q2_ref, g2_ref, b2_ref, out_ref,
             *, count):
    mean2 = s2_ref[...] / count
    var2 = q2_ref[...] / count - mean2 * mean2
    scale2 = g2_ref[...] * jax.lax.rsqrt(var2 + EPS)
    shift2 = b2_ref[...] - mean2 * scale2
    z = jnp.maximum(ymax_ref[0] * scale2, ymin_ref[0] * scale2) + shift2
    out_ref[0] = jnp.maximum(z, 0.0).T


def _sc_gather(table, idx):
    v, dimw = table.shape
    bsz = idx.shape[0]
    nw = 32
    per_w = bsz // nw
    chunk = 128
    nchunk = per_w // chunk

    @functools.partial(
        pl.kernel,
        out_type=jax.ShapeDtypeStruct((bsz, dimw), jnp.float32),
        mesh=plsc.VectorSubcoreMesh(core_axis_name="c", subcore_axis_name="s"),
        scratch_types=[
            pltpu.VMEM((2, chunk), jnp.int32),
            pltpu.VMEM((2, chunk, dimw), jnp.float32),
            pltpu.SemaphoreType.DMA,
            pltpu.SemaphoreType.DMA,
        ],
    )
    def k(table_hbm, idx_hbm, out_hbm, idx_v, rows_v, gsem, osem):
        wid = lax.axis_index("s") * 2 + lax.axis_index("c")
        base = wid * per_w

        def body(i, carry):
            slot = lax.rem(i, 2)
            pltpu.sync_copy(idx_hbm.at[pl.ds(base + i * chunk, chunk)],
                            idx_v.at[slot])
            cp = pltpu.async_copy(table_hbm.at[idx_v.at[slot]],
                                  rows_v.at[slot], gsem)
            cp.wait()
            ocp = pltpu.async_copy(rows_v.at[slot],
                                   out_hbm.at[pl.ds(base + i * chunk, chunk)],
                                   osem)
            ocp.wait()
            return carry

        lax.fori_loop(0, nchunk, body, 0)

    return k(table, idx)


def _mlp_bn_max(gg4, h3, W2, gamma1, beta1, gamma2, beta2, b):
    _, _, m, _ = gg4.shape
    rows = b * K_NEIGH * m
    count = float(rows)
    bm = b * m
    MT = 256
    nmt = m // MT
    vec = lambda v: v.reshape(1, 128)
    s1, q1 = pl.pallas_call(
        lambda *refs: _ks_body(*refs, mt=MT),
        grid=(b, nmt),
        in_specs=[
            pl.BlockSpec((1, K_NEIGH, MT, 128), lambda bi, mi: (bi, 0, mi, 0)),
            pl.BlockSpec((1, MT, 128), lambda bi, mi: (bi, mi, 0)),
        ],
        out_specs=[
            pl.BlockSpec((1, 128), lambda bi, mi: (0, 0)),
            pl.BlockSpec((1, 128), lambda bi, mi: (0, 0)),
        ],
        out_shape=[
            jax.ShapeDtypeStruct((1, 128), jnp.float32),
            jax.ShapeDtypeStruct((1, 128), jnp.float32),
        ],
    )(gg4, h3)

    ymax, ymin, s2, q2 = pl.pallas_call(
        lambda *refs: _kb_body(*refs, count=count, mt=MT),
        grid=(b, nmt),
        in_specs=[
            pl.BlockSpec((1, K_NEIGH, MT, 128), lambda bi, mi: (bi, 0, mi, 0)),
            pl.BlockSpec((1, MT, 128), lambda bi, mi: (bi, mi, 0)),
            pl.BlockSpec((1, 128), lambda bi, mi: (0, 0)),
            pl.BlockSpec((1, 128), lambda bi, mi: (0, 0)),
            pl.BlockSpec((1, 128), lambda bi, mi: (0, 0)),
            pl.BlockSpec((1, 128), lambda bi, mi: (0, 0)),
            pl.BlockSpec((128, 128), lambda bi, mi: (0, 0)),
        ],
        out_specs=[
            pl.BlockSpec((MT, 128), lambda bi, mi: (bi * nmt + mi, 0)),
            pl.BlockSpec((MT, 128), lambda bi, mi: (bi * nmt + mi, 0)),
            pl.BlockSpec((1, 128), lambda bi, mi: (0, 0)),
            pl.BlockSpec((1, 128), lambda bi, mi: (0, 0)),
        ],
        out_shape=[
            jax.ShapeDtypeStruct((bm, 128), jnp.float32),
            jax.ShapeDtypeStruct((bm, 128), jnp.float32),
            jax.ShapeDtypeStruct((1, 128), jnp.float32),
            jax.ShapeDtypeStruct((1, 128), jnp.float32),
        ],
    )(gg4, h3, s1, q1, vec(gamma1), vec(beta1), W2)
    ymax3 = ymax.reshape(b, m, 128)
    ymin3 = ymin.reshape(b, m, 128)
    MT2 = 512
    out = pl.pallas_call(
        lambda *refs: _kc_body(*refs, count=count),
        grid=(b, m // MT2),
        in_specs=[
            pl.BlockSpec((1, MT2, 128), lambda bi, mi: (bi, mi, 0)),
            pl.BlockSpec((1, MT2, 128), lambda bi, mi: (bi, mi, 0)),
            pl.BlockSpec((1, 128), lambda bi, mi: (0, 0)),
            pl.BlockSpec((1, 128), lambda bi, mi: (0, 0)),
            pl.BlockSpec((1, 128), lambda bi, mi: (0, 0)),
            pl.BlockSpec((1, 128), lambda bi, mi: (0, 0)),
        ],
        out_specs=pl.BlockSpec((1, 128, MT2), lambda bi, mi: (bi, 0, mi)),
        out_shape=jax.ShapeDtypeStruct((b, 128, m), jnp.float32),
    )(ymax3, ymin3, s2, q2, vec(gamma2), vec(beta2))
    return out


def kernel(xyz, feat, npoints, W1, gamma1, beta1, W2, gamma2, beta2):
    b, n, _ = xyz.shape
    c = feat.shape[1]
    m = M_OUT
    k = K_NEIGH
    sample_idx = (jnp.tile(jnp.arange(m, dtype=jnp.int32)[None, :], (b, 1))
                  + (jnp.asarray(npoints).astype(jnp.int32) - m))
    new_xyz = xyz[:, :m, :]

    knn_idx = _knn(xyz, b, n, m)

    w1a_t = jnp.transpose(W1[:, :c + 3])
    wd_t = jnp.transpose(W1[:, c + 3:]) - w1a_t
    G = _proj(feat, xyz, w1a_t, b, n, 512)
    H = _proj(feat[:, :, :m], xyz[:, :m, :], wd_t, b, m, 512)

    idx_flat = (knn_idx + (jnp.arange(b, dtype=jnp.int32) * n)[:, None, None])
    gg = _sc_gather(G.reshape(b * n, 128), idx_flat.reshape(-1))
    gg4 = gg.reshape(b, K_NEIGH, m, 128)

    out_feat = _mlp_bn_max(gg4, H, W2, gamma1, beta1, gamma2, beta2, b)
    return new_xyz, out_feat, sample_idx.astype(jnp.int64)


# speedup vs baseline: 20.5024x; 1.0360x over previous
import functools

import jax
import jax.numpy as jnp
from jax import lax
from jax.experimental import pallas as pl
from jax.experimental.pallas import tpu as pltpu
from jax.experimental.pallas import tpu_sc as plsc

EPS = 1e-5
K_NEIGH = 16
M_OUT = 2048
BIG = 3.0e38


def _knn_body(p_ref, qt_ref, idx_ref, *, tq, n):
    ngrp = n // 8
    p = p_ref[0]
    qt = qt_ref[0]
    dot = jnp.dot(p, qt, preferred_element_type=jnp.float32)
    qq = jnp.sum(qt * qt, axis=0, keepdims=True)
    pp = jnp.sum(p * p, axis=1, keepdims=True)
    d = (qq - 2.0 * dot) + pp
    d3 = d.reshape(ngrp, 8, tq)
    gmin = jnp.min(d3, axis=1)
    giota = jax.lax.broadcasted_iota(jnp.int32, (ngrp, tq), 0)
    gsel = []
    for _ in range(K_NEIGH):
        v = jnp.min(gmin, axis=0, keepdims=True)
        gi = jnp.min(jnp.where(gmin == v, giota, ngrp), axis=0, keepdims=True)
        gsel.append(gi)
        gmin = jnp.where(giota == gi, BIG, gmin)
    nbit = max(1, (ngrp - 1).bit_length())
    cands = []
    for t in range(K_NEIGH):
        gi = gsel[t]
        x = d3
        for bit in reversed(range(nbit)):
            half = x.shape[0] // 2
            take_hi = ((gi >> bit) & 1) == 1
            x = jnp.where(take_hi[:, None, :], x[half:], x[:half])
        cands.append(x)
    cand = jnp.concatenate(cands, axis=0)
    gsel_arr = jnp.concatenate(gsel, axis=0)
    sub = jax.lax.broadcasted_iota(jnp.int32, (K_NEIGH, 8, tq), 1)
    gidx = gsel_arr[:, None, :] * 8 + sub
    flat = cand.reshape(K_NEIGH * 8, tq)
    fidx = gidx.reshape(K_NEIGH * 8, tq)
    outs = []
    for _ in range(K_NEIGH):
        v = jnp.min(flat, axis=0, keepdims=True)
        ji = jnp.min(jnp.where(flat == v, fidx, jnp.int32(n)), axis=0,
                     keepdims=True)
        outs.append(ji)
        flat = jnp.where(fidx == ji, BIG, flat)
    idx_ref[0] = jnp.concatenate(outs, axis=0)


def _knn(xyz, b, n, m):
    TQ = 128
    xyz_t = jnp.transpose(xyz[:, :m, :], (0, 2, 1))
    return pl.pallas_call(
        lambda *refs: _knn_body(*refs, tq=TQ, n=n),
        grid=(b, m // TQ),
        in_specs=[
            pl.BlockSpec((1, n, 3), lambda bi, mi: (bi, 0, 0)),
            pl.BlockSpec((1, 3, TQ), lambda bi, mi: (bi, 0, mi)),
        ],
        out_specs=pl.BlockSpec((1, K_NEIGH, TQ), lambda bi, mi: (bi, 0, mi)),
        out_shape=jax.ShapeDtypeStruct((b, K_NEIGH, m), jnp.int32),
    )(xyz, xyz_t)


def _proj_body(feat_ref, xyz_ref, w_ref, g_ref):
    ft = feat_ref[0].T
    fc = jnp.concatenate([ft, xyz_ref[0]], axis=1)
    g_ref[0] = jnp.dot(fc, w_ref[...], preferred_element_type=jnp.float32)


def _proj(feat, xyz, w, b, n, tn):
    c = feat.shape[1]
    return pl.pallas_call(
        _proj_body,
        grid=(b, n // tn),
        in_specs=[
            pl.BlockSpec((1, c, tn), lambda bi, ni: (bi, 0, ni)),
            pl.BlockSpec((1, tn, 3), lambda bi, ni: (bi, ni, 0)),
            pl.BlockSpec((c + 3, 128), lambda bi, ni: (0, 0)),
        ],
        out_specs=pl.BlockSpec((1, tn, 128), lambda bi, ni: (bi, ni, 0)),
        out_shape=jax.ShapeDtypeStruct((b, n, 128), jnp.float32),
    )(feat, xyz, w)


def _ks_body(gg_ref, h_ref, s1_ref, q1_ref, *, mt):
    first = (pl.program_id(0) == 0) & (pl.program_id(1) == 0)
    x1 = gg_ref[0] + h_ref[0][None, :, :]

    @pl.when(first)
    def _init():
        s1_ref[...] = jnp.zeros_like(s1_ref)
        q1_ref[...] = jnp.zeros_like(q1_ref)

    s1_ref[...] += jnp.sum(x1, axis=(0, 1)).reshape(1, 128)
    q1_ref[...] += jnp.sum(x1 * x1, axis=(0, 1)).reshape(1, 128)


def _kb_body(gg_ref, h_ref, s1_ref, q1_ref, g1_ref, b1_ref, w2_ref,
             ymax_ref, ymin_ref, s2_ref, q2_ref, *, count, mt):
    first = (pl.program_id(0) == 0) & (pl.program_id(1) == 0)
    mean1 = s1_ref[...] / count
    var1 = q1_ref[...] / count - mean1 * mean1
    scale1 = g1_ref[...] * jax.lax.rsqrt(var1 + EPS)
    shift1 = b1_ref[...] - mean1 * scale1
    x1 = (gg_ref[0] + h_ref[0][None, :, :]).reshape(K_NEIGH * mt, 128)
    h = jnp.maximum(x1 * scale1 + shift1, 0.0)
    x2 = jnp.dot(h, w2_ref[...].T, preferred_element_type=jnp.float32)

    @pl.when(first)
    def _init():
        s2_ref[...] = jnp.zeros_like(s2_ref)
        q2_ref[...] = jnp.zeros_like(q2_ref)

    s2_ref[...] += jnp.sum(x2, axis=0, keepdims=True)
    q2_ref[...] += jnp.sum(x2 * x2, axis=0, keepdims=True)
    x2r = x2.reshape(K_NEIGH, mt, 128)
    ymax_ref[...] = jnp.max(x2r, axis=0)
    ymin_ref[...] = jnp.min(x2r, axis=0)


def _kc_body(ymax_ref, ymin_ref, s2_ref, q2_ref, g2_ref, b2_ref, out_ref,
             *, count):
    mean2 = s2_ref[...] / count
    var2 = q2_ref[...] / count - mean2 * mean2
    scale2 = g2_ref[...] * jax.lax.rsqrt(var2 + EPS)
    shift2 = b2_ref[...] - mean2 * scale2
    z = jnp.maximum(ymax_ref[0] * scale2, ymin_ref[0] * scale2) + shift2
    out_ref[0] = jnp.maximum(z, 0.0).T


def _sc_gather(table, idx):
    v, dimw = table.shape
    bsz = idx.shape[0]
    nw = 32
    per_w = bsz // nw
    chunk = 256
    nchunk = per_w // chunk

    @functools.partial(
        pl.kernel,
        out_type=jax.ShapeDtypeStruct((bsz, dimw), jnp.float32),
        mesh=plsc.VectorSubcoreMesh(core_axis_name="c", subcore_axis_name="s"),
        scratch_types=[
            pltpu.VMEM((per_w,), jnp.int32),
            pltpu.VMEM((2, chunk, dimw), jnp.float32),
            pltpu.SemaphoreType.DMA,
            pltpu.SemaphoreType.DMA,
        ],
    )
    def k(table_hbm, idx_hbm, out_hbm, idx_v, rows_v, gsem, osem):
        wid = lax.axis_index("s") * 2 + lax.axis_index("c")
        base = wid * per_w
        pltpu.sync_copy(idx_hbm.at[pl.ds(base, per_w)], idx_v)

        def start_gather(i):
            pltpu.async_copy(
                table_hbm.at[idx_v.at[pl.ds(i * chunk, chunk)]],
                rows_v.at[lax.rem(i, 2)], gsem)

        def drain(sem, ref_slot):
            pltpu.make_async_copy(
                out_hbm.at[pl.ds(base, chunk)], ref_slot, sem).wait()

        start_gather(0)

        def body(i, carry):
            slot = lax.rem(i, 2)
            drain(gsem, rows_v.at[slot])

            @pl.when(i + 1 < nchunk)
            def _next():
                @pl.when(i >= 1)
                def _wb_done():
                    drain(osem, rows_v.at[1 - slot])
                start_gather(i + 1)

            pltpu.async_copy(rows_v.at[slot],
                             out_hbm.at[pl.ds(base + i * chunk, chunk)],
                             osem)
            return carry

        lax.fori_loop(0, nchunk, body, 0)
        drain(osem, rows_v.at[0])
        drain(osem, rows_v.at[1])

    return k(table, idx)


def _mlp_bn_max(gg4, h3, W2, gamma1, beta1, gamma2, beta2, b):
    _, _, m, _ = gg4.shape
    rows = b * K_NEIGH * m
    count = float(rows)
    bm = b * m
    MT = 256
    nmt = m // MT
    vec = lambda v: v.reshape(1, 128)
    s1, q1 = pl.pallas_call(
        lambda *refs: _ks_body(*refs, mt=MT),
        grid=(b, nmt),
        in_specs=[
            pl.BlockSpec((1, K_NEIGH, MT, 128), lambda bi, mi: (bi, 0, mi, 0)),
            pl.BlockSpec((1, MT, 128), lambda bi, mi: (bi, mi, 0)),
        ],
        out_specs=[
            pl.BlockSpec((1, 128), lambda bi, mi: (0, 0)),
            pl.BlockSpec((1, 128), lambda bi, mi: (0, 0)),
        ],
        out_shape=[
            jax.ShapeDtypeStruct((1, 128), jnp.float32),
            jax.ShapeDtypeStruct((1, 128), jnp.float32),
        ],
    )(gg4, h3)

    ymax, ymin, s2, q2 = pl.pallas_call(
        lambda *refs: _kb_body(*refs, count=count, mt=MT),
        grid=(b, nmt),
        in_specs=[
            pl.BlockSpec((1, K_NEIGH, MT, 128), lambda bi, mi: (bi, 0, mi, 0)),
            pl.BlockSpec((1, MT, 128), lambda bi, mi: (bi, mi, 0)),
            pl.BlockSpec((1, 128), lambda bi, mi: (0, 0)),
            pl.BlockSpec((1, 128), lambda bi, mi: (0, 0)),
            pl.BlockSpec((1, 128), lambda bi, mi: (0, 0)),
            pl.BlockSpec((1, 128), lambda bi, mi: (0, 0)),
            pl.BlockSpec((128, 128), lambda bi, mi: (0, 0)),
        ],
        out_specs=[
            pl.BlockSpec((MT, 128), lambda bi, mi: (bi * nmt + mi, 0)),
            pl.BlockSpec((MT, 128), lambda bi, mi: (bi * nmt + mi, 0)),
            pl.BlockSpec((1, 128), lambda bi, mi: (0, 0)),
            pl.BlockSpec((1, 128), lambda bi, mi: (0, 0)),
        ],
        out_shape=[
            jax.ShapeDtypeStruct((bm, 128), jnp.float32),
            jax.ShapeDtypeStruct((bm, 128), jnp.float32),
            jax.ShapeDtypeStruct((1, 128), jnp.float32),
            jax.ShapeDtypeStruct((1, 128), jnp.float32),
        ],
    )(gg4, h3, s1, q1, vec(gamma1), vec(beta1), W2)
    ymax3 = ymax.reshape(b, m, 128)
    ymin3 = ymin.reshape(b, m, 128)
    MT2 = 512
    out = pl.pallas_call(
        lambda *refs: _kc_body(*refs, count=count),
        grid=(b, m // MT2),
        in_specs=[
            pl.BlockSpec((1, MT2, 128), lambda bi, mi: (bi, mi, 0)),
            pl.BlockSpec((1, MT2, 128), lambda bi, mi: (bi, mi, 0)),
            pl.BlockSpec((1, 128), lambda bi, mi: (0, 0)),
            pl.BlockSpec((1, 128), lambda bi, mi: (0, 0)),
            pl.BlockSpec((1, 128), lambda bi, mi: (0, 0)),
            pl.BlockSpec((1, 128), lambda bi, mi: (0, 0)),
        ],
        out_specs=pl.BlockSpec((1, 128, MT2), lambda bi, mi: (bi, 0, mi)),
        out_shape=jax.ShapeDtypeStruct((b, 128, m), jnp.float32),
    )(ymax3, ymin3, s2, q2, vec(gamma2), vec(beta2))
    return out


def kernel(xyz, feat, npoints, W1, gamma1, beta1, W2, gamma2, beta2):
    b, n, _ = xyz.shape
    c = feat.shape[1]
    m = M_OUT
    k = K_NEIGH
    sample_idx = (jnp.tile(jnp.arange(m, dtype=jnp.int32)[None, :], (b, 1))
                  + (jnp.asarray(npoints).astype(jnp.int32) - m))
    new_xyz = xyz[:, :m, :]

    knn_idx = _knn(xyz, b, n, m)

    w1a_t = jnp.transpose(W1[:, :c + 3])
    wd_t = jnp.transpose(W1[:, c + 3:]) - w1a_t
    G = _proj(feat, xyz, w1a_t, b, n, 512)
    H = _proj(feat[:, :, :m], xyz[:, :m, :], wd_t, b, m, 512)

    idx_flat = (knn_idx + (jnp.arange(b, dtype=jnp.int32) * n)[:, None, None])
    gg = _sc_gather(G.reshape(b * n, 128), idx_flat.reshape(-1))
    gg4 = gg.reshape(b, K_NEIGH, m, 128)

    out_feat = _mlp_bn_max(gg4, H, W2, gamma1, beta1, gamma2, beta2, b)
    return new_xyz, out_feat, sample_idx.astype(jnp.int64)


# speedup vs baseline: 23.3898x; 1.1408x over previous
import functools

import jax
import jax.numpy as jnp
from jax import lax
from jax.experimental import pallas as pl
from jax.experimental.pallas import tpu as pltpu
from jax.experimental.pallas import tpu_sc as plsc

EPS = 1e-5
K_NEIGH = 16
M_OUT = 2048
BIG = 3.0e38


def _knn_body(p_ref, qt_ref, idx_ref, *, tq, n):
    ngrp = n // 8
    p = p_ref[0]
    qt = qt_ref[0]
    dot = jnp.dot(p, qt, preferred_element_type=jnp.float32)
    qq = jnp.sum(qt * qt, axis=0, keepdims=True)
    pp = jnp.sum(p * p, axis=1, keepdims=True)
    d = (qq - 2.0 * dot) + pp
    d3 = d.reshape(ngrp, 8, tq)
    gmin = jnp.min(d3, axis=1)
    nch = ngrp // 8
    g2 = gmin.reshape(nch, 8, tq)
    l2 = jnp.min(g2, axis=1)
    c_iota = jax.lax.broadcasted_iota(jnp.int32, (nch, tq), 0)
    csel = []
    for _ in range(K_NEIGH):
        v = jnp.min(l2, axis=0, keepdims=True)
        ci = jnp.min(jnp.where(l2 == v, c_iota, nch), axis=0, keepdims=True)
        csel.append(ci)
        l2 = jnp.where(c_iota == ci, BIG, l2)
    nbit2 = max(1, (nch - 1).bit_length())
    c2 = []
    for t in range(K_NEIGH):
        ci = csel[t]
        x = g2
        for bit in reversed(range(nbit2)):
            half = x.shape[0] // 2
            take_hi = ((ci >> bit) & 1) == 1
            x = jnp.where(take_hi[:, None, :], x[half:], x[:half])
        c2.append(x)
    cand2 = jnp.concatenate(c2, axis=0).reshape(K_NEIGH * 8, tq)
    sub2 = jax.lax.broadcasted_iota(jnp.int32, (K_NEIGH, 8, tq), 1)
    pid = (jnp.concatenate(csel, axis=0)[:, None, :] * 8
           + sub2).reshape(K_NEIGH * 8, tq)
    gsel = []
    for _ in range(K_NEIGH):
        v = jnp.min(cand2, axis=0, keepdims=True)
        gi = jnp.min(jnp.where(cand2 == v, pid, ngrp), axis=0, keepdims=True)
        gsel.append(gi)
        cand2 = jnp.where(pid == gi, BIG, cand2)
    nbit = max(1, (ngrp - 1).bit_length())
    cands = []
    for t in range(K_NEIGH):
        gi = gsel[t]
        x = d3
        for bit in reversed(range(nbit)):
            half = x.shape[0] // 2
            take_hi = ((gi >> bit) & 1) == 1
            x = jnp.where(take_hi[:, None, :], x[half:], x[:half])
        cands.append(x)
    cand = jnp.concatenate(cands, axis=0)
    gsel_arr = jnp.concatenate(gsel, axis=0)
    sub = jax.lax.broadcasted_iota(jnp.int32, (K_NEIGH, 8, tq), 1)
    gidx = gsel_arr[:, None, :] * 8 + sub
    flat = cand.reshape(K_NEIGH * 8, tq)
    fidx = gidx.reshape(K_NEIGH * 8, tq)
    outs = []
    for _ in range(K_NEIGH):
        v = jnp.min(flat, axis=0, keepdims=True)
        ji = jnp.min(jnp.where(flat == v, fidx, jnp.int32(n)), axis=0,
                     keepdims=True)
        outs.append(ji)
        flat = jnp.where(fidx == ji, BIG, flat)
    base = pl.program_id(0) * n
    idx_ref[0] = jnp.concatenate(outs, axis=0) + base


def _knn(xyz, b, n, m):
    TQ = 128
    xyz_t = jnp.transpose(xyz[:, :m, :], (0, 2, 1))
    return pl.pallas_call(
        lambda *refs: _knn_body(*refs, tq=TQ, n=n),
        grid=(b, m // TQ),
        in_specs=[
            pl.BlockSpec((1, n, 3), lambda bi, mi: (bi, 0, 0)),
            pl.BlockSpec((1, 3, TQ), lambda bi, mi: (bi, 0, mi)),
        ],
        out_specs=pl.BlockSpec((1, K_NEIGH, TQ), lambda bi, mi: (bi, 0, mi)),
        out_shape=jax.ShapeDtypeStruct((b, K_NEIGH, m), jnp.int32),
    )(xyz, xyz_t)


def _proj_body(feat_ref, xyz_ref, w_ref, g_ref):
    ft = feat_ref[0].T
    fc = jnp.concatenate([ft, xyz_ref[0]], axis=1)
    g_ref[0] = jnp.dot(fc, w_ref[...], preferred_element_type=jnp.float32)


def _proj(feat, xyz, w, b, n, tn):
    c = feat.shape[1]
    return pl.pallas_call(
        _proj_body,
        grid=(b, n // tn),
        in_specs=[
            pl.BlockSpec((1, c, tn), lambda bi, ni: (bi, 0, ni)),
            pl.BlockSpec((1, tn, 3), lambda bi, ni: (bi, ni, 0)),
            pl.BlockSpec((c + 3, 128), lambda bi, ni: (0, 0)),
        ],
        out_specs=pl.BlockSpec((1, tn, 128), lambda bi, ni: (bi, ni, 0)),
        out_shape=jax.ShapeDtypeStruct((b, n, 128), jnp.float32),
    )(feat, xyz, w)


def _ks_body(gg_ref, h_ref, s1_ref, q1_ref, *, mt):
    first = (pl.program_id(0) == 0) & (pl.program_id(1) == 0)
    x1 = gg_ref[0] + h_ref[0][None, :, :]

    @pl.when(first)
    def _init():
        s1_ref[...] = jnp.zeros_like(s1_ref)
        q1_ref[...] = jnp.zeros_like(q1_ref)

    s1_ref[...] += jnp.sum(x1, axis=(0, 1)).reshape(1, 128)
    q1_ref[...] += jnp.sum(x1 * x1, axis=(0, 1)).reshape(1, 128)


def _kb_body(gg_ref, h_ref, s1_ref, q1_ref, g1_ref, b1_ref, w2_ref,
             ymax_ref, ymin_ref, s2_ref, q2_ref, *, count, mt):
    first = (pl.program_id(0) == 0) & (pl.program_id(1) == 0)
    mean1 = s1_ref[...] / count
    var1 = q1_ref[...] / count - mean1 * mean1
    scale1 = g1_ref[...] * jax.lax.rsqrt(var1 + EPS)
    shift1 = b1_ref[...] - mean1 * scale1
    x1 = (gg_ref[0] + h_ref[0][None, :, :]).reshape(K_NEIGH * mt, 128)
    h = jnp.maximum(x1 * scale1 + shift1, 0.0)
    x2 = jnp.dot(h, w2_ref[...].T, preferred_element_type=jnp.float32)

    @pl.when(first)
    def _init():
        s2_ref[...] = jnp.zeros_like(s2_ref)
        q2_ref[...] = jnp.zeros_like(q2_ref)

    s2_ref[...] += jnp.sum(x2, axis=0, keepdims=True)
    q2_ref[...] += jnp.sum(x2 * x2, axis=0, keepdims=True)
    x2r = x2.reshape(K_NEIGH, mt, 128)
    ymax_ref[...] = jnp.max(x2r, axis=0)
    ymin_ref[...] = jnp.min(x2r, axis=0)


def _kc_body(ymax_ref, ymin_ref, s2_ref, q2_ref, g2_ref, b2_ref, out_ref,
             *, count):
    mean2 = s2_ref[...] / count
    var2 = q2_ref[...] / count - mean2 * mean2
    scale2 = g2_ref[...] * jax.lax.rsqrt(var2 + EPS)
    shift2 = b2_ref[...] - mean2 * scale2
    z = jnp.maximum(ymax_ref[0] * scale2, ymin_ref[0] * scale2) + shift2
    out_ref[0] = jnp.maximum(z, 0.0).T


def _sc_gather(table, idx):
    v, dimw = table.shape
    bsz = idx.shape[0]
    nw = 32
    per_w = bsz // nw
    chunk = 256
    nchunk = per_w // chunk

    @functools.partial(
        pl.kernel,
        out_type=jax.ShapeDtypeStruct((bsz, dimw), jnp.float32),
        mesh=plsc.VectorSubcoreMesh(core_axis_name="c", subcore_axis_name="s"),
        scratch_types=[
            pltpu.VMEM((per_w,), jnp.int32),
            pltpu.VMEM((2, chunk, dimw), jnp.float32),
            pltpu.SemaphoreType.DMA,
            pltpu.SemaphoreType.DMA,
        ],
    )
    def k(table_hbm, idx_hbm, out_hbm, idx_v, rows_v, gsem, osem):
        wid = lax.axis_index("s") * 2 + lax.axis_index("c")
        base = wid * per_w
        pltpu.sync_copy(idx_hbm.at[pl.ds(base, per_w)], idx_v)

        def start_gather(i):
            pltpu.async_copy(
                table_hbm.at[idx_v.at[pl.ds(i * chunk, chunk)]],
                rows_v.at[lax.rem(i, 2)], gsem)

        def drain(sem, ref_slot):
            pltpu.make_async_copy(
                out_hbm.at[pl.ds(base, chunk)], ref_slot, sem).wait()

        start_gather(0)

        def body(i, carry):
            slot = lax.rem(i, 2)
            drain(gsem, rows_v.at[slot])

            @pl.when(i + 1 < nchunk)
            def _next():
                @pl.when(i >= 1)
                def _wb_done():
                    drain(osem, rows_v.at[1 - slot])
                start_gather(i + 1)

            pltpu.async_copy(rows_v.at[slot],
                             out_hbm.at[pl.ds(base + i * chunk, chunk)],
                             osem)
            return carry

        lax.fori_loop(0, nchunk, body, 0)
        drain(osem, rows_v.at[0])
        drain(osem, rows_v.at[1])

    return k(table, idx)


def _mlp_bn_max(gg4, h3, W2, gamma1, beta1, gamma2, beta2, b):
    _, _, m, _ = gg4.shape
    rows = b * K_NEIGH * m
    count = float(rows)
    bm = b * m
    MT = 256
    nmt = m // MT
    vec = lambda v: v.reshape(1, 128)
    s1, q1 = pl.pallas_call(
        lambda *refs: _ks_body(*refs, mt=MT),
        grid=(b, nmt),
        in_specs=[
            pl.BlockSpec((1, K_NEIGH, MT, 128), lambda bi, mi: (bi, 0, mi, 0)),
            pl.BlockSpec((1, MT, 128), lambda bi, mi: (bi, mi, 0)),
        ],
        out_specs=[
            pl.BlockSpec((1, 128), lambda bi, mi: (0, 0)),
            pl.BlockSpec((1, 128), lambda bi, mi: (0, 0)),
        ],
        out_shape=[
            jax.ShapeDtypeStruct((1, 128), jnp.float32),
            jax.ShapeDtypeStruct((1, 128), jnp.float32),
        ],
    )(gg4, h3)

    ymax, ymin, s2, q2 = pl.pallas_call(
        lambda *refs: _kb_body(*refs, count=count, mt=MT),
        grid=(b, nmt),
        in_specs=[
            pl.BlockSpec((1, K_NEIGH, MT, 128), lambda bi, mi: (bi, 0, mi, 0)),
            pl.BlockSpec((1, MT, 128), lambda bi, mi: (bi, mi, 0)),
            pl.BlockSpec((1, 128), lambda bi, mi: (0, 0)),
            pl.BlockSpec((1, 128), lambda bi, mi: (0, 0)),
            pl.BlockSpec((1, 128), lambda bi, mi: (0, 0)),
            pl.BlockSpec((1, 128), lambda bi, mi: (0, 0)),
            pl.BlockSpec((128, 128), lambda bi, mi: (0, 0)),
        ],
        out_specs=[
            pl.BlockSpec((MT, 128), lambda bi, mi: (bi * nmt + mi, 0)),
            pl.BlockSpec((MT, 128), lambda bi, mi: (bi * nmt + mi, 0)),
            pl.BlockSpec((1, 128), lambda bi, mi: (0, 0)),
            pl.BlockSpec((1, 128), lambda bi, mi: (0, 0)),
        ],
        out_shape=[
            jax.ShapeDtypeStruct((bm, 128), jnp.float32),
            jax.ShapeDtypeStruct((bm, 128), jnp.float32),
            jax.ShapeDtypeStruct((1, 128), jnp.float32),
            jax.ShapeDtypeStruct((1, 128), jnp.float32),
        ],
    )(gg4, h3, s1, q1, vec(gamma1), vec(beta1), W2)
    ymax3 = ymax.reshape(b, m, 128)
    ymin3 = ymin.reshape(b, m, 128)
    MT2 = 512
    out = pl.pallas_call(
        lambda *refs: _kc_body(*refs, count=count),
        grid=(b, m // MT2),
        in_specs=[
            pl.BlockSpec((1, MT2, 128), lambda bi, mi: (bi, mi, 0)),
            pl.BlockSpec((1, MT2, 128), lambda bi, mi: (bi, mi, 0)),
            pl.BlockSpec((1, 128), lambda bi, mi: (0, 0)),
            pl.BlockSpec((1, 128), lambda bi, mi: (0, 0)),
            pl.BlockSpec((1, 128), lambda bi, mi: (0, 0)),
            pl.BlockSpec((1, 128), lambda bi, mi: (0, 0)),
        ],
        out_specs=pl.BlockSpec((1, 128, MT2), lambda bi, mi: (bi, 0, mi)),
        out_shape=jax.ShapeDtypeStruct((b, 128, m), jnp.float32),
    )(ymax3, ymin3, s2, q2, vec(gamma2), vec(beta2))
    return out


def kernel(xyz, feat, npoints, W1, gamma1, beta1, W2, gamma2, beta2):
    b, n, _ = xyz.shape
    c = feat.shape[1]
    m = M_OUT
    k = K_NEIGH
    sample_idx = (jnp.tile(jnp.arange(m, dtype=jnp.int32)[None, :], (b, 1))
                  + (jnp.asarray(npoints).astype(jnp.int32) - m))
    new_xyz = xyz[:, :m, :]

    knn_idx = _knn(xyz, b, n, m)

    w1a_t = jnp.transpose(W1[:, :c + 3])
    wd_t = jnp.transpose(W1[:, c + 3:]) - w1a_t
    G = _proj(feat, xyz, w1a_t, b, n, 512)
    H = _proj(feat[:, :, :m], xyz[:, :m, :], wd_t, b, m, 512)

    gg = _sc_gather(G.reshape(b * n, 128), knn_idx.reshape(-1))
    gg4 = gg.reshape(b, K_NEIGH, m, 128)

    out_feat = _mlp_bn_max(gg4, H, W2, gamma1, beta1, gamma2, beta2, b)
    return new_xyz, out_feat, sample_idx.astype(jnp.int64)


# speedup vs baseline: 23.9509x; 1.0240x over previous
import functools

import jax
import jax.numpy as jnp
from jax import lax
from jax.experimental import pallas as pl
from jax.experimental.pallas import tpu as pltpu
from jax.experimental.pallas import tpu_sc as plsc

EPS = 1e-5
K_NEIGH = 16
M_OUT = 2048
BIG = 3.0e38


def _knn_body(p_ref, qt_ref, idx_ref, *, tq, n):
    ngrp = n // 8
    p = p_ref[0]
    qt = qt_ref[0]
    dot = jnp.dot(p, qt, preferred_element_type=jnp.float32)
    qq = jnp.sum(qt * qt, axis=0, keepdims=True)
    pp = jnp.sum(p * p, axis=1, keepdims=True)
    d = (qq - 2.0 * dot) + pp
    d3 = d.reshape(ngrp, 8, tq)
    gmin = jnp.min(d3, axis=1)
    nch = ngrp // 8
    g2 = gmin.reshape(nch, 8, tq)
    l2 = jnp.min(g2, axis=1)
    c_iota = jax.lax.broadcasted_iota(jnp.int32, (nch, tq), 0)
    csel = []
    for _ in range(K_NEIGH):
        v = jnp.min(l2, axis=0, keepdims=True)
        ci = jnp.min(jnp.where(l2 == v, c_iota, nch), axis=0, keepdims=True)
        csel.append(ci)
        l2 = jnp.where(c_iota == ci, BIG, l2)
    nbit2 = max(1, (nch - 1).bit_length())
    c2 = []
    for t in range(K_NEIGH):
        ci = csel[t]
        x = g2
        for bit in reversed(range(nbit2)):
            half = x.shape[0] // 2
            take_hi = ((ci >> bit) & 1) == 1
            x = jnp.where(take_hi[:, None, :], x[half:], x[:half])
        c2.append(x)
    cand2 = jnp.concatenate(c2, axis=0).reshape(K_NEIGH * 8, tq)
    sub2 = jax.lax.broadcasted_iota(jnp.int32, (K_NEIGH, 8, tq), 1)
    pid = (jnp.concatenate(csel, axis=0)[:, None, :] * 8
           + sub2).reshape(K_NEIGH * 8, tq)
    gsel = []
    for _ in range(K_NEIGH):
        v = jnp.min(cand2, axis=0, keepdims=True)
        gi = jnp.min(jnp.where(cand2 == v, pid, ngrp), axis=0, keepdims=True)
        gsel.append(gi)
        cand2 = jnp.where(pid == gi, BIG, cand2)
    nbit = max(1, (ngrp - 1).bit_length())
    cands = []
    for t in range(K_NEIGH):
        gi = gsel[t]
        x = d3
        for bit in reversed(range(nbit)):
            half = x.shape[0] // 2
            take_hi = ((gi >> bit) & 1) == 1
            x = jnp.where(take_hi[:, None, :], x[half:], x[:half])
        cands.append(x)
    cand = jnp.concatenate(cands, axis=0)
    gsel_arr = jnp.concatenate(gsel, axis=0)
    sub = jax.lax.broadcasted_iota(jnp.int32, (K_NEIGH, 8, tq), 1)
    gidx = gsel_arr[:, None, :] * 8 + sub
    flat = cand.reshape(K_NEIGH * 8, tq)
    fidx = gidx.reshape(K_NEIGH * 8, tq)
    outs = []
    for _ in range(K_NEIGH):
        v = jnp.min(flat, axis=0, keepdims=True)
        ji = jnp.min(jnp.where(flat == v, fidx, jnp.int32(n)), axis=0,
                     keepdims=True)
        outs.append(ji)
        flat = jnp.where(fidx == ji, BIG, flat)
    base = pl.program_id(0) * n
    idx_ref[0] = jnp.concatenate(outs, axis=0) + base


def _knn(xyz, b, n, m):
    TQ = 256
    xyz_t = jnp.transpose(xyz[:, :m, :], (0, 2, 1))
    return pl.pallas_call(
        lambda *refs: _knn_body(*refs, tq=TQ, n=n),
        grid=(b, m // TQ),
        in_specs=[
            pl.BlockSpec((1, n, 3), lambda bi, mi: (bi, 0, 0)),
            pl.BlockSpec((1, 3, TQ), lambda bi, mi: (bi, 0, mi)),
        ],
        out_specs=pl.BlockSpec((1, K_NEIGH, TQ), lambda bi, mi: (bi, 0, mi)),
        out_shape=jax.ShapeDtypeStruct((b, K_NEIGH, m), jnp.int32),
    )(xyz, xyz_t)


def _proj_body(feat_ref, xyz_ref, w_ref, g_ref):
    ft = feat_ref[0].T
    fc = jnp.concatenate([ft, xyz_ref[0]], axis=1)
    g_ref[0] = jnp.dot(fc, w_ref[...], preferred_element_type=jnp.float32)


def _proj(feat, xyz, w, b, n, tn):
    c = feat.shape[1]
    return pl.pallas_call(
        _proj_body,
        grid=(b, n // tn),
        in_specs=[
            pl.BlockSpec((1, c, tn), lambda bi, ni: (bi, 0, ni)),
            pl.BlockSpec((1, tn, 3), lambda bi, ni: (bi, ni, 0)),
            pl.BlockSpec((c + 3, 128), lambda bi, ni: (0, 0)),
        ],
        out_specs=pl.BlockSpec((1, tn, 128), lambda bi, ni: (bi, ni, 0)),
        out_shape=jax.ShapeDtypeStruct((b, n, 128), jnp.float32),
    )(feat, xyz, w)


def _ks_body(gg_ref, h_ref, s1_ref, q1_ref, *, mt):
    first = (pl.program_id(0) == 0) & (pl.program_id(1) == 0)
    x1 = gg_ref[0] + h_ref[0][None, :, :]

    @pl.when(first)
    def _init():
        s1_ref[...] = jnp.zeros_like(s1_ref)
        q1_ref[...] = jnp.zeros_like(q1_ref)

    s1_ref[...] += jnp.sum(x1, axis=(0, 1)).reshape(1, 128)
    q1_ref[...] += jnp.sum(x1 * x1, axis=(0, 1)).reshape(1, 128)


def _kb_body(gg_ref, h_ref, s1_ref, q1_ref, g1_ref, b1_ref, w2_ref,
             ymax_ref, ymin_ref, s2_ref, q2_ref, *, count, mt):
    first = (pl.program_id(0) == 0) & (pl.program_id(1) == 0)
    mean1 = s1_ref[...] / count
    var1 = q1_ref[...] / count - mean1 * mean1
    scale1 = g1_ref[...] * jax.lax.rsqrt(var1 + EPS)
    shift1 = b1_ref[...] - mean1 * scale1
    x1 = (gg_ref[0] + h_ref[0][None, :, :]).reshape(K_NEIGH * mt, 128)
    h = jnp.maximum(x1 * scale1 + shift1, 0.0)
    x2 = jnp.dot(h, w2_ref[...].T, preferred_element_type=jnp.float32)

    @pl.when(first)
    def _init():
        s2_ref[...] = jnp.zeros_like(s2_ref)
        q2_ref[...] = jnp.zeros_like(q2_ref)

    s2_ref[...] += jnp.sum(x2, axis=0, keepdims=True)
    q2_ref[...] += jnp.sum(x2 * x2, axis=0, keepdims=True)
    x2r = x2.reshape(K_NEIGH, mt, 128)
    ymax_ref[...] = jnp.max(x2r, axis=0)
    ymin_ref[...] = jnp.min(x2r, axis=0)


def _kc_body(ymax_ref, ymin_ref, s2_ref, q2_ref, g2_ref, b2_ref, out_ref,
             *, count):
    mean2 = s2_ref[...] / count
    var2 = q2_ref[...] / count - mean2 * mean2
    scale2 = g2_ref[...] * jax.lax.rsqrt(var2 + EPS)
    shift2 = b2_ref[...] - mean2 * scale2
    z = jnp.maximum(ymax_ref[0] * scale2, ymin_ref[0] * scale2) + shift2
    out_ref[0] = jnp.maximum(z, 0.0).T


def _sc_gather(table, idx):
    v, dimw = table.shape
    bsz = idx.shape[0]
    nw = 32
    per_w = bsz // nw
    chunk = 256
    nchunk = per_w // chunk

    @functools.partial(
        pl.kernel,
        out_type=jax.ShapeDtypeStruct((bsz, dimw), jnp.float32),
        mesh=plsc.VectorSubcoreMesh(core_axis_name="c", subcore_axis_name="s"),
        scratch_types=[
            pltpu.VMEM((per_w,), jnp.int32),
            pltpu.VMEM((2, chunk, dimw), jnp.float32),
            pltpu.SemaphoreType.DMA,
            pltpu.SemaphoreType.DMA,
        ],
    )
    def k(table_hbm, idx_hbm, out_hbm, idx_v, rows_v, gsem, osem):
        wid = lax.axis_index("s") * 2 + lax.axis_index("c")
        base = wid * per_w
        pltpu.sync_copy(idx_hbm.at[pl.ds(base, per_w)], idx_v)

        def start_gather(i):
            pltpu.async_copy(
                table_hbm.at[idx_v.at[pl.ds(i * chunk, chunk)]],
                rows_v.at[lax.rem(i, 2)], gsem)

        def drain(sem, ref_slot):
            pltpu.make_async_copy(
                out_hbm.at[pl.ds(base, chunk)], ref_slot, sem).wait()

        start_gather(0)

        def body(i, carry):
            slot = lax.rem(i, 2)
            drain(gsem, rows_v.at[slot])

            @pl.when(i + 1 < nchunk)
            def _next():
                @pl.when(i >= 1)
                def _wb_done():
                    drain(osem, rows_v.at[1 - slot])
                start_gather(i + 1)

            pltpu.async_copy(rows_v.at[slot],
                             out_hbm.at[pl.ds(base + i * chunk, chunk)],
                             osem)
            return carry

        lax.fori_loop(0, nchunk, body, 0)
        drain(osem, rows_v.at[0])
        drain(osem, rows_v.at[1])

    return k(table, idx)


def _mlp_bn_max(gg4, h3, W2, gamma1, beta1, gamma2, beta2, b):
    _, _, m, _ = gg4.shape
    rows = b * K_NEIGH * m
    count = float(rows)
    bm = b * m
    MT = 256
    nmt = m // MT
    vec = lambda v: v.reshape(1, 128)
    s1, q1 = pl.pallas_call(
        lambda *refs: _ks_body(*refs, mt=MT),
        grid=(b, nmt),
        in_specs=[
            pl.BlockSpec((1, K_NEIGH, MT, 128), lambda bi, mi: (bi, 0, mi, 0)),
            pl.BlockSpec((1, MT, 128), lambda bi, mi: (bi, mi, 0)),
        ],
        out_specs=[
            pl.BlockSpec((1, 128), lambda bi, mi: (0, 0)),
            pl.BlockSpec((1, 128), lambda bi, mi: (0, 0)),
        ],
        out_shape=[
            jax.ShapeDtypeStruct((1, 128), jnp.float32),
            jax.ShapeDtypeStruct((1, 128), jnp.float32),
        ],
    )(gg4, h3)

    ymax, ymin, s2, q2 = pl.pallas_call(
        lambda *refs: _kb_body(*refs, count=count, mt=MT),
        grid=(b, nmt),
        in_specs=[
            pl.BlockSpec((1, K_NEIGH, MT, 128), lambda bi, mi: (bi, 0, mi, 0)),
            pl.BlockSpec((1, MT, 128), lambda bi, mi: (bi, mi, 0)),
            pl.BlockSpec((1, 128), lambda bi, mi: (0, 0)),
            pl.BlockSpec((1, 128), lambda bi, mi: (0, 0)),
            pl.BlockSpec((1, 128), lambda bi, mi: (0, 0)),
            pl.BlockSpec((1, 128), lambda bi, mi: (0, 0)),
            pl.BlockSpec((128, 128), lambda bi, mi: (0, 0)),
        ],
        out_specs=[
            pl.BlockSpec((MT, 128), lambda bi, mi: (bi * nmt + mi, 0)),
            pl.BlockSpec((MT, 128), lambda bi, mi: (bi * nmt + mi, 0)),
            pl.BlockSpec((1, 128), lambda bi, mi: (0, 0)),
            pl.BlockSpec((1, 128), lambda bi, mi: (0, 0)),
        ],
        out_shape=[
            jax.ShapeDtypeStruct((bm, 128), jnp.float32),
            jax.ShapeDtypeStruct((bm, 128), jnp.float32),
            jax.ShapeDtypeStruct((1, 128), jnp.float32),
            jax.ShapeDtypeStruct((1, 128), jnp.float32),
        ],
    )(gg4, h3, s1, q1, vec(gamma1), vec(beta1), W2)
    ymax3 = ymax.reshape(b, m, 128)
    ymin3 = ymin.reshape(b, m, 128)
    MT2 = 512
    out = pl.pallas_call(
        lambda *refs: _kc_body(*refs, count=count),
        grid=(b, m // MT2),
        in_specs=[
            pl.BlockSpec((1, MT2, 128), lambda bi, mi: (bi, mi, 0)),
            pl.BlockSpec((1, MT2, 128), lambda bi, mi: (bi, mi, 0)),
            pl.BlockSpec((1, 128), lambda bi, mi: (0, 0)),
            pl.BlockSpec((1, 128), lambda bi, mi: (0, 0)),
            pl.BlockSpec((1, 128), lambda bi, mi: (0, 0)),
            pl.BlockSpec((1, 128), lambda bi, mi: (0, 0)),
        ],
        out_specs=pl.BlockSpec((1, 128, MT2), lambda bi, mi: (bi, 0, mi)),
        out_shape=jax.ShapeDtypeStruct((b, 128, m), jnp.float32),
    )(ymax3, ymin3, s2, q2, vec(gamma2), vec(beta2))
    return out


def kernel(xyz, feat, npoints, W1, gamma1, beta1, W2, gamma2, beta2):
    b, n, _ = xyz.shape
    c = feat.shape[1]
    m = M_OUT
    k = K_NEIGH
    sample_idx = (jnp.tile(jnp.arange(m, dtype=jnp.int32)[None, :], (b, 1))
                  + (jnp.asarray(npoints).astype(jnp.int32) - m))
    new_xyz = xyz[:, :m, :]

    knn_idx = _knn(xyz, b, n, m)

    w1a_t = jnp.transpose(W1[:, :c + 3])
    wd_t = jnp.transpose(W1[:, c + 3:]) - w1a_t
    G = _proj(feat, xyz, w1a_t, b, n, 512)
    H = _proj(feat[:, :, :m], xyz[:, :m, :], wd_t, b, m, 512)

    gg = _sc_gather(G.reshape(b * n, 128), knn_idx.reshape(-1))
    gg4 = gg.reshape(b, K_NEIGH, m, 128)

    out_feat = _mlp_bn_max(gg4, H, W2, gamma1, beta1, gamma2, beta2, b)
    return new_xyz, out_feat, sample_idx.astype(jnp.int64)
